# Initial kernel scaffold; baseline (speedup 1.0000x reference)
import jax
import jax.numpy as jnp
from jax.experimental import pallas as pl


def kernel(x, adj_a, adj_b, Wa0, ba0, Wa1, ba1, Wb0, bb0, Wb1, bb1, Wm, bm, Wo, bo):
    raise NotImplementedError("write your pallas kernel here")


# speedup vs baseline: 3.1697x; 3.1697x over previous
import functools

import jax
import jax.numpy as jnp
from jax import lax
from jax.experimental import pallas as pl
from jax.experimental.pallas import tpu as pltpu
from jax.experimental.pallas import tpu_sc as plsc

N = 10000
E = 320000
D = 128

NC = 2
NS = 16
NW = NC * NS
EPW = E // NW
CH = 80
NCHUNK = EPW // CH
NPAD = 10112
RPT = NPAD // NS

_sc_mesh = plsc.VectorSubcoreMesh(
    core_axis_name="c", subcore_axis_name="s", num_cores=NC, num_subcores=NS)


@functools.partial(
    pl.kernel,
    out_type=jax.ShapeDtypeStruct((NC * NPAD, D), jnp.float32),
    mesh=_sc_mesh,
    scratch_types=[
        pltpu.VMEM((CH,), jnp.int32),
        pltpu.VMEM((CH,), jnp.int32),
        pltpu.VMEM((CH, D), jnp.float32),
        pltpu.VMEM_SHARED((NPAD, D), jnp.float32),
        pltpu.SemaphoreType.DMA,
    ],
)
def _spmv_sc(h_hbm, src_hbm, dst_hbm, zero_hbm, out_hbm,
             src_v, dst_v, rows_v, acc_sh, sem):
    cid = lax.axis_index("c")
    sid = lax.axis_index("s")
    wid = cid * NS + sid
    pltpu.sync_copy(zero_hbm.at[pl.ds(sid * RPT, RPT)],
                    acc_sh.at[pl.ds(sid * RPT, RPT)])
    plsc.subcore_barrier()
    ebase = wid * EPW

    def body(i, _):
        b = ebase + i * CH
        pltpu.sync_copy(src_hbm.at[pl.ds(b, CH)], src_v)
        pltpu.sync_copy(dst_hbm.at[pl.ds(b, CH)], dst_v)
        pltpu.async_copy(h_hbm.at[src_v], rows_v, sem).wait()
        pltpu.sync_copy(rows_v, acc_sh.at[dst_v], add=True)
        return ()

    lax.fori_loop(0, NCHUNK, body, ())
    plsc.subcore_barrier()
    pltpu.sync_copy(acc_sh.at[pl.ds(sid * RPT, RPT)],
                    out_hbm.at[pl.ds(cid * NPAD + sid * RPT, RPT)])


def _spmv(h, src, dst, zeros):
    p = _spmv_sc(h, src, dst, zeros)
    return p[:N], p[NPAD:NPAD + N]


_RBLK = 2000


def _make_dense_body(term_sizes, relu):
    def body(*refs):
        o_ref = refs[-1]
        b_ref = refs[-2]
        idx = 0
        acc = None
        for npart in term_sizes:
            xs = refs[idx:idx + npart]
            w_ref = refs[idx + npart]
            idx += npart + 1
            xsum = xs[0][...]
            for r in xs[1:]:
                xsum = xsum + r[...]
            t = jnp.dot(xsum, w_ref[...], preferred_element_type=jnp.float32)
            acc = t if acc is None else acc + t
        acc = acc + b_ref[...]
        if relu:
            acc = jnp.maximum(acc, 0.0)
        o_ref[...] = acc
    return body


def _dense(terms, b, relu):
    term_sizes = tuple(len(parts) for parts, _ in terms)
    in_specs = []
    args = []
    for parts, w in terms:
        for p in parts:
            in_specs.append(pl.BlockSpec((_RBLK, p.shape[1]), lambda i: (i, 0)))
            args.append(p)
        in_specs.append(
            pl.BlockSpec((w.shape[0], w.shape[1]), lambda i: (0, 0)))
        args.append(w)
    b2 = b[None, :]
    in_specs.append(pl.BlockSpec((1, b2.shape[1]), lambda i: (0, 0)))
    args.append(b2)
    h = terms[0][1].shape[1]
    return pl.pallas_call(
        _make_dense_body(term_sizes, relu),
        grid=(N // _RBLK,),
        in_specs=in_specs,
        out_specs=pl.BlockSpec((_RBLK, h), lambda i: (i, 0)),
        out_shape=jax.ShapeDtypeStruct((N, h), jnp.float32),
    )(*args)


def kernel(x, adj_a, adj_b, Wa0, ba0, Wa1, ba1, Wb0, bb0, Wb1, bb1,
           Wm, bm, Wo, bo):
    zeros = jnp.zeros((NPAD, D), jnp.float32)
    src_a, dst_a = adj_a[1], adj_a[0]
    src_b, dst_b = adj_b[1], adj_b[0]
    Wm0, Wm1, Wm2 = Wm[:D], Wm[D:2 * D], Wm[2 * D:]
    Wo_a, Wo_b = Wo[:D], Wo[D:]

    ha0 = _dense([([x], Wa0)], ba0, relu=True)
    pa0, pa1 = _spmv(ha0, src_a, dst_a, zeros)
    ha1 = _dense([([pa0, pa1], Wa1)], ba1, relu=True)
    qa0, qa1 = _spmv(ha1, src_a, dst_a, zeros)

    hb0 = _dense([([x], Wb0)], bb0, relu=True)
    pb0, pb1 = _spmv(hb0, src_b, dst_b, zeros)
    hb1 = _dense([([pb0, pb1], Wb1)], bb1, relu=True)
    qb0, qb1 = _spmv(hb1, src_b, dst_b, zeros)
    xb = _dense([([hb0], Wm0), ([hb1], Wm1), ([qb0, qb1], Wm2)], bm, relu=True)

    out = _dense([([qa0, qa1], Wo_a), ([xb], Wo_b)], bo, relu=False)
    return out


# speedup vs baseline: 8.2645x; 2.6073x over previous
import functools

import jax
import jax.numpy as jnp
from jax import lax
from jax.experimental import pallas as pl
from jax.experimental.pallas import tpu as pltpu
from jax.experimental.pallas import tpu_sc as plsc

N = 10000
E = 320000
D = 128

NC = 2
NS = 16
NW = NC * NS
EPW = E // NW
CH = 128
NCHUNK = EPW // CH
CHT = EPW - NCHUNK * CH
NPAD = 10112
RPT = NPAD // NS

@functools.cache
def _spmv_sc_build():
    mesh = plsc.VectorSubcoreMesh(
        core_axis_name="c", subcore_axis_name="s",
        num_cores=NC, num_subcores=NS)
    return pl.kernel(
        _spmv_sc,
        out_type=jax.ShapeDtypeStruct((NC * NPAD, D), jnp.float32),
        mesh=mesh,
        scratch_types=[
        pltpu.VMEM((EPW,), jnp.int32),
        pltpu.VMEM((CH,), jnp.int32),
        pltpu.VMEM((CH,), jnp.int32),
        pltpu.VMEM((CH, D), jnp.float32),
        pltpu.VMEM((CH, D), jnp.float32),
        pltpu.VMEM_SHARED((NPAD, D), jnp.float32),
        pltpu.SemaphoreType.DMA,
        pltpu.SemaphoreType.DMA,
        pltpu.SemaphoreType.DMA,
        pltpu.SemaphoreType.DMA,
        ],
    )


def _spmv_sc(h_hbm, src_hbm, dst_hbm, zero_hbm, out_hbm,
             src_v, dst0_v, dst1_v, rows0_v, rows1_v, acc_sh,
             gsem0, gsem1, dsem0, dsem1):
    cid = lax.axis_index("c")
    sid = lax.axis_index("s")
    wid = cid * NS + sid
    ebase = wid * EPW
    dst_v = (dst0_v, dst1_v)
    rows_v = (rows0_v, rows1_v)
    gsem = (gsem0, gsem1)
    dsem = (dsem0, dsem1)

    pltpu.sync_copy(src_hbm.at[pl.ds(ebase, EPW)], src_v)
    pltpu.sync_copy(zero_hbm.at[pl.ds(sid * RPT, RPT)],
                    acc_sh.at[pl.ds(sid * RPT, RPT)])
    plsc.subcore_barrier()

    def start(i, b):
        pltpu.async_copy(dst_hbm.at[pl.ds(ebase + i * CH, CH)],
                         dst_v[b], dsem[b])
        pltpu.async_copy(h_hbm.at[src_v.at[pl.ds(i * CH, CH)]],
                         rows_v[b], gsem[b])

    def finish(i, b):
        pltpu.make_async_copy(dst_hbm.at[pl.ds(ebase + i * CH, CH)],
                              dst_v[b], dsem[b]).wait()
        pltpu.make_async_copy(h_hbm.at[src_v.at[pl.ds(i * CH, CH)]],
                              rows_v[b], gsem[b]).wait()
        pltpu.sync_copy(rows_v[b], acc_sh.at[dst_v[b]], add=True)

    start(0, 0)
    start(1, 1)

    def body(k, _):
        i = 2 * k
        finish(i, 0)
        start(i + 2, 0)
        finish(i + 1, 1)
        start(i + 3, 1)
        return ()

    lax.fori_loop(0, NCHUNK // 2 - 1, body, ())
    finish(NCHUNK - 2, 0)
    finish(NCHUNK - 1, 1)

    tb = ebase + NCHUNK * CH
    pltpu.async_copy(dst_hbm.at[pl.ds(tb, CHT)], dst_v[0].at[pl.ds(0, CHT)],
                     dsem[0])
    pltpu.async_copy(h_hbm.at[src_v.at[pl.ds(NCHUNK * CH, CHT)]],
                     rows_v[0].at[pl.ds(0, CHT)], gsem[0])
    pltpu.make_async_copy(dst_hbm.at[pl.ds(tb, CHT)],
                          dst_v[0].at[pl.ds(0, CHT)], dsem[0]).wait()
    pltpu.make_async_copy(h_hbm.at[src_v.at[pl.ds(NCHUNK * CH, CHT)]],
                          rows_v[0].at[pl.ds(0, CHT)], gsem[0]).wait()
    pltpu.sync_copy(rows_v[0].at[pl.ds(0, CHT)],
                    acc_sh.at[dst_v[0].at[pl.ds(0, CHT)]], add=True)

    plsc.subcore_barrier()
    pltpu.sync_copy(acc_sh.at[pl.ds(sid * RPT, RPT)],
                    out_hbm.at[pl.ds(cid * NPAD + sid * RPT, RPT)])


def _spmv(h, src, dst, zeros):
    p = _spmv_sc_build()(h, src, dst, zeros)
    return p[:N], p[NPAD:NPAD + N]


_RBLK = 2000


def _make_dense_body(term_sizes, relu):
    def body(*refs):
        o_ref = refs[-1]
        b_ref = refs[-2]
        idx = 0
        acc = None
        for npart in term_sizes:
            xs = refs[idx:idx + npart]
            w_ref = refs[idx + npart]
            idx += npart + 1
            xsum = xs[0][...]
            for r in xs[1:]:
                xsum = xsum + r[...]
            t = jnp.dot(xsum, w_ref[...], preferred_element_type=jnp.float32)
            acc = t if acc is None else acc + t
        acc = acc + b_ref[...]
        if relu:
            acc = jnp.maximum(acc, 0.0)
        o_ref[...] = acc
    return body


def _dense(terms, b, relu):
    term_sizes = tuple(len(parts) for parts, _ in terms)
    in_specs = []
    args = []
    for parts, w in terms:
        for p in parts:
            in_specs.append(pl.BlockSpec((_RBLK, p.shape[1]), lambda i: (i, 0)))
            args.append(p)
        in_specs.append(
            pl.BlockSpec((w.shape[0], w.shape[1]), lambda i: (0, 0)))
        args.append(w)
    b2 = b[None, :]
    in_specs.append(pl.BlockSpec((1, b2.shape[1]), lambda i: (0, 0)))
    args.append(b2)
    h = terms[0][1].shape[1]
    return pl.pallas_call(
        _make_dense_body(term_sizes, relu),
        grid=(N // _RBLK,),
        in_specs=in_specs,
        out_specs=pl.BlockSpec((_RBLK, h), lambda i: (i, 0)),
        out_shape=jax.ShapeDtypeStruct((N, h), jnp.float32),
    )(*args)


def kernel(x, adj_a, adj_b, Wa0, ba0, Wa1, ba1, Wb0, bb0, Wb1, bb1,
           Wm, bm, Wo, bo):
    zeros = jnp.zeros((NPAD, D), jnp.float32)
    src_a, dst_a = adj_a[1], adj_a[0]
    src_b, dst_b = adj_b[1], adj_b[0]
    Wm0, Wm1, Wm2 = Wm[:D], Wm[D:2 * D], Wm[2 * D:]
    Wo_a, Wo_b = Wo[:D], Wo[D:]

    ha0 = _dense([([x], Wa0)], ba0, relu=True)
    pa0, pa1 = _spmv(ha0, src_a, dst_a, zeros)
    ha1 = _dense([([pa0, pa1], Wa1)], ba1, relu=True)
    qa0, qa1 = _spmv(ha1, src_a, dst_a, zeros)

    hb0 = _dense([([x], Wb0)], bb0, relu=True)
    pb0, pb1 = _spmv(hb0, src_b, dst_b, zeros)
    hb1 = _dense([([pb0, pb1], Wb1)], bb1, relu=True)
    qb0, qb1 = _spmv(hb1, src_b, dst_b, zeros)
    xb = _dense([([hb0], Wm0), ([hb1], Wm1), ([qb0, qb1], Wm2)], bm, relu=True)

    out = _dense([([qa0, qa1], Wo_a), ([xb], Wo_b)], bo, relu=False)
    return out
